# Initial kernel scaffold; baseline (speedup 1.0000x reference)
import jax
import jax.numpy as jnp
from jax.experimental import pallas as pl


def kernel(x, pos, edge_index, edge_index_bond, edge_attr_bond, triplet_kj, triplet_ji, batch, params):
    raise NotImplementedError("write your pallas kernel here")


# speedup vs baseline: 1.0000x; 1.0000x over previous
import jax
import jax.numpy as jnp
import numpy as np
from jax.experimental import pallas as pl
from jax.experimental.pallas import tpu as pltpu

N = 10000; E = 320000; EB = 40000; T = 640000
H = 128; IN = 128; RB = 32; AB = 16; L = 4; G = 64


def _rbf(d, rmin, rmax, nb):
    c = jnp.linspace(rmin, rmax, nb, dtype=jnp.float32)
    delta = (rmax - rmin) / max(nb - 1, 1)
    gamma = 1.0 / (2.0 * (delta ** 2 + 1e-12))
    diff = d[:, None] - c[None, :]
    return jnp.exp(-gamma * diff * diff)


def _ln(x, g, b):
    mu = x.mean(-1, keepdims=True)
    var = x.var(-1, keepdims=True)
    return (x - mu) / jnp.sqrt(var + 1e-5) * g + b


def _mlp(x, W1, b1, W2, b2):
    return jax.nn.silu(x @ W1 + b1) @ W2 + b2


def _head_body(gg_ref, w1_ref, b1_ref, w2_ref, b2_ref, o_ref):
    h = jnp.dot(gg_ref[...], w1_ref[...], preferred_element_type=jnp.float32)
    h = h + b1_ref[...][None, :]
    h = h * jax.nn.sigmoid(h)
    o = jnp.dot(h, w2_ref[...], preferred_element_type=jnp.float32)
    o_ref[...] = o + b2_ref[...][None, :]


def _head(gg, w1, b1, w2, b2):
    return pl.pallas_call(
        _head_body,
        out_shape=jax.ShapeDtypeStruct((G, 1), jnp.float32),
    )(gg, w1, b1, w2, b2)


def kernel(x, pos, edge_index, edge_index_bond, edge_attr_bond, triplet_kj, triplet_ji, batch, params):
    P = params
    src = edge_index[0]; dst = edge_index[1]
    rel = pos[dst] - pos[src]
    dist = jnp.sqrt(jnp.sum(rel * rel, -1) + 1e-12)
    edir = rel / dist[:, None]
    erbf = _rbf(dist, 0.0, 5.0, RB)
    s = x @ P['Win'] + P['bin']
    v = jnp.zeros((N, H, 3), dtype=jnp.float32)
    bsrc = edge_index_bond[0]; bdst = edge_index_bond[1]
    ee = edge_attr_bond @ P['bgWe'] + P['bgbe']
    bm = jax.nn.relu(s[bsrc] + ee)
    bagg = jnp.zeros((N, H)).at[bdst].add(bm)
    s = s + _mlp(s + bagg, P['bgW1'], P['bgb1'], P['bgW2'], P['bgb2'])
    s = jax.nn.silu(_ln(s, P['bglng'], P['bglnb']))
    cosang = jnp.clip(jnp.sum(edir[triplet_kj] * edir[triplet_ji], -1), -1.0 + 1e-7, 1.0 - 1e-7)
    af = _rbf(jnp.arccos(cosang), 0.0, float(np.pi), AB)
    vn = jnp.broadcast_to(P['vn0'], (G, H))
    for i in range(L):
        s = s + _mlp(vn, P['vnaW1'], P['vnab1'], P['vnaW2'], P['vnab2'])[batch]
        t_in = jnp.concatenate([erbf[triplet_kj], erbf[triplet_ji], af], -1)
        t_msg = _mlp(t_in, P['t1W%d' % i], P['t1b%d' % i], P['t2W%d' % i], P['t2b%d' % i])
        angle_agg = jnp.zeros((E, H)).at[triplet_ji].add(t_msg)
        msg_in = jnp.concatenate([s[src], s[dst], erbf, angle_agg], -1)
        msg = _mlp(msg_in, P['m1W%d' % i], P['m1b%d' % i], P['m2W%d' % i], P['m2b%d' % i])
        m_s = msg[:, :H]; m_vc = msg[:, H:]
        agg_s = jnp.zeros((N, H)).at[dst].add(m_s)
        agg_v = jnp.zeros((N, H, 3)).at[dst].add(m_vc[:, :, None] * edir[:, None, :])
        v_norm = jnp.sqrt(jnp.sum(v * v, -1) + 1e-12)
        ctx = jnp.concatenate([s, agg_s, v_norm], -1)
        s = s + _mlp(ctx, P['u1W%d' % i], P['u1b%d' % i], P['u2W%d' % i], P['u2b%d' % i])
        gate = jax.nn.sigmoid(_mlp(ctx, P['g1W%d' % i], P['g1b%d' % i], P['g2W%d' % i], P['g2b%d' % i]))
        v = v + gate[:, :, None] * agg_v
        s = jax.nn.silu(_ln(s, P['lng%d' % i], P['lnb%d' % i]))
        vn = vn + _mlp(jax.ops.segment_sum(s, batch, G), P['vnbW1'], P['vnbb1'], P['vnbW2'], P['vnbb2'])
    sum_g = jax.ops.segment_sum(s, batch, G)
    cnt = jax.ops.segment_sum(jnp.ones((N, 1), dtype=jnp.float32), batch, G)
    mean_g = sum_g / jnp.maximum(cnt, 1.0)
    gg = jnp.concatenate([sum_g, mean_g], -1)
    return _head(gg, P['h1W'], P['h1b'], P['h2W'], P['h2b'])


# speedup vs baseline: 1.0048x; 1.0048x over previous
import functools
import jax
import jax.numpy as jnp
import numpy as np
from jax import lax
from jax.experimental import pallas as pl
from jax.experimental.pallas import tpu as pltpu
from jax.experimental.pallas import tpu_sc as plsc

N = 10000; E = 320000; EB = 40000; T = 640000
H = 128; IN = 128; RB = 32; AB = 16; L = 4; G = 64

_ASIN_C = [4.2163199048e-2, 2.4181311049e-2, 4.5470025998e-2,
           7.4953002686e-2, 1.6666752422e-1]


def _acos(x):
    t = jnp.abs(x)
    w = jnp.sqrt(jnp.maximum((1.0 - t) * 0.5, 0.0))
    u = jnp.where(t <= 0.5, t, w)
    z = u * u
    p = jnp.full_like(z, _ASIN_C[0])
    for cf in _ASIN_C[1:]:
        p = p * z + cf
    asin_u = u + u * z * p
    r = jnp.where(t <= 0.5, np.float32(np.pi / 2) - asin_u, 2.0 * asin_u)
    return jnp.where(x >= 0.0, r, np.float32(np.pi) - r)


def _sigmoid(x):
    return jax.nn.sigmoid(x)


def _silu(x):
    return x * _sigmoid(x)


def _sc_gather(table, idx, gb=80):
    B, = idx.shape
    R, D = table.shape
    NW = 32
    chunk = NW * gb
    Bp = ((B + chunk - 1) // chunk) * chunk
    if Bp != B:
        idx = jnp.concatenate([idx, jnp.zeros((Bp - B,), idx.dtype)])
    bt = Bp // NW
    nit = bt // gb
    mesh = plsc.VectorSubcoreMesh(core_axis_name="c", subcore_axis_name="s")

    @functools.partial(
        pl.kernel, mesh=mesh,
        out_type=jax.ShapeDtypeStruct((Bp, D), jnp.float32),
        scratch_types=[pltpu.VMEM((gb,), jnp.int32),
                       pltpu.VMEM((gb, D), jnp.float32),
                       pltpu.SemaphoreType.DMA],
    )
    def k(table_hbm, idx_hbm, out_hbm, idx_v, rows_v, sem):
        wid = lax.axis_index("s") * 2 + lax.axis_index("c")
        base = wid * bt

        def body(it, carry):
            off = base + it * gb
            pltpu.sync_copy(idx_hbm.at[pl.ds(off, gb)], idx_v)
            pltpu.async_copy(table_hbm.at[idx_v], rows_v, sem).wait()
            pltpu.sync_copy(rows_v, out_hbm.at[pl.ds(off, gb)])
            return carry

        lax.fori_loop(0, nit, body, 0)

    out = k(table, idx)
    return out[:B] if Bp != B else out


def _mlp_body(has2, act1, has_resid, k_lo, k_hi, *refs):
    if has2 and has_resid:
        x_ref, w1_ref, b1_ref, w2_ref, b2_ref, r_ref, o_ref = refs
    elif has2:
        x_ref, w1_ref, b1_ref, w2_ref, b2_ref, o_ref = refs
        r_ref = None
    elif has_resid:
        x_ref, w1_ref, b1_ref, r_ref, o_ref = refs
    else:
        x_ref, w1_ref, b1_ref, o_ref = refs
        r_ref = None
    x = x_ref[...]
    if k_lo != 0 or k_hi != x.shape[1]:
        x = x[:, k_lo:k_hi]
    h = jnp.dot(x, w1_ref[...], preferred_element_type=jnp.float32) + b1_ref[...]
    if act1 == 'silu':
        h = _silu(h)
    elif act1 == 'relu':
        h = jnp.maximum(h, 0.0)
    if has2:
        h = jnp.dot(h, w2_ref[...], preferred_element_type=jnp.float32) + b2_ref[...]
    if r_ref is not None:
        h = h + r_ref[...]
    o_ref[...] = h


def _mlp_call(X, W1, b1, W2=None, b2=None, act1=None, resid=None,
              k_lo=0, k_hi=None, mb=2000):
    M, K = X.shape
    if k_hi is None:
        k_hi = K
    Hout = W2.shape[1] if W2 is not None else W1.shape[1]
    H1 = W1.shape[1]
    grid = M // mb
    assert grid * mb == M, (M, mb)
    ins = [X, W1.reshape(W1.shape), b1.reshape(1, H1)]
    specs = [pl.BlockSpec((mb, K), lambda i: (i, 0)),
             pl.BlockSpec(W1.shape, lambda i: (0, 0)),
             pl.BlockSpec((1, H1), lambda i: (0, 0))]
    if W2 is not None:
        ins += [W2, b2.reshape(1, Hout)]
        specs += [pl.BlockSpec(W2.shape, lambda i: (0, 0)),
                  pl.BlockSpec((1, Hout), lambda i: (0, 0))]
    if resid is not None:
        ins.append(resid)
        specs.append(pl.BlockSpec((mb, Hout), lambda i: (i, 0)))
    body = functools.partial(_mlp_body, W2 is not None, act1,
                             resid is not None, k_lo, k_hi)
    return pl.pallas_call(
        body,
        grid=(grid,),
        in_specs=specs,
        out_specs=pl.BlockSpec((mb, Hout), lambda i: (i, 0)),
        out_shape=jax.ShapeDtypeStruct((M, Hout), jnp.float32),
    )(*ins)


def _geo_body(pp_ref, o_ref):
    pp = pp_ref[...]
    rel = pp[:, 4:7] - pp[:, 0:3]
    d2 = jnp.sum(rel * rel, axis=1, keepdims=True) + 1e-12
    dist = jnp.sqrt(d2)
    edir = rel / dist
    delta = 5.0 / (RB - 1)
    gamma = 1.0 / (2.0 * (delta ** 2 + 1e-12))
    c = jax.lax.broadcasted_iota(jnp.int32, (1, RB), 1).astype(jnp.float32) * np.float32(delta)
    diff = dist - c
    erbf = jnp.exp(-gamma * diff * diff)
    pad = jnp.zeros((pp.shape[0], 93), jnp.float32)
    o_ref[...] = jnp.concatenate([erbf, edir, pad], axis=1)


def _geo_call(pp, mb=2000):
    grid = E // mb
    return pl.pallas_call(
        _geo_body,
        grid=(grid,),
        in_specs=[pl.BlockSpec((mb, 8), lambda i: (i, 0))],
        out_specs=pl.BlockSpec((mb, 128), lambda i: (i, 0)),
        out_shape=jax.ShapeDtypeStruct((E, 128), jnp.float32),
    )(pp)


def _tripfeat_body(gk_ref, gj_ref, o_ref):
    gk = gk_ref[...]
    gj = gj_ref[...]
    cos = jnp.sum(gk[:, 32:35] * gj[:, 32:35], axis=1, keepdims=True)
    cos = jnp.clip(cos, -1.0 + 1e-7, 1.0 - 1e-7)
    theta = _acos(cos)
    delta = float(np.pi) / (AB - 1)
    gamma = 1.0 / (2.0 * (delta ** 2 + 1e-12))
    c = jax.lax.broadcasted_iota(jnp.int32, (1, AB), 1).astype(jnp.float32) * np.float32(delta)
    diff = theta - c
    af = jnp.exp(-gamma * diff * diff)
    o_ref[...] = jnp.concatenate([gk[:, 0:32], gj[:, 0:32], af], axis=1)


def _tripfeat_call(gk, gj, mb=2000):
    grid = T // mb
    return pl.pallas_call(
        _tripfeat_body,
        grid=(grid,),
        in_specs=[pl.BlockSpec((mb, 128), lambda i: (i, 0)),
                  pl.BlockSpec((mb, 128), lambda i: (i, 0))],
        out_specs=pl.BlockSpec((mb, 80), lambda i: (i, 0)),
        out_shape=jax.ShapeDtypeStruct((T, 80), jnp.float32),
    )(gk, gj)


def _edge_body(pgs_ref, pgd_ref, q_ref, a_ref, geo_ref, wd_ref, b1_ref,
               w2_ref, b2_ref, o_ref):
    adot = jnp.dot(a_ref[...], wd_ref[...], preferred_element_type=jnp.float32)
    h1 = pgs_ref[...] + pgd_ref[...] + q_ref[...] + adot + b1_ref[...]
    m = jnp.dot(_silu(h1), w2_ref[...], preferred_element_type=jnp.float32) + b2_ref[...]
    geo = geo_ref[...]
    m_s = m[:, :H]
    m_vc = m[:, H:]
    o_ref[0] = m_s
    o_ref[1] = m_vc * geo[:, 32:33]
    o_ref[2] = m_vc * geo[:, 33:34]
    o_ref[3] = m_vc * geo[:, 34:35]


def _edge_call(pgs, pgd, q, a, geo, wd, b1, w2, b2, mb=2000):
    grid = E // mb
    return pl.pallas_call(
        _edge_body,
        grid=(grid,),
        in_specs=[pl.BlockSpec((mb, H), lambda i: (i, 0)),
                  pl.BlockSpec((mb, H), lambda i: (i, 0)),
                  pl.BlockSpec((mb, H), lambda i: (i, 0)),
                  pl.BlockSpec((mb, H), lambda i: (i, 0)),
                  pl.BlockSpec((mb, 128), lambda i: (i, 0)),
                  pl.BlockSpec((H, H), lambda i: (0, 0)),
                  pl.BlockSpec((1, H), lambda i: (0, 0)),
                  pl.BlockSpec((H, 2 * H), lambda i: (0, 0)),
                  pl.BlockSpec((1, 2 * H), lambda i: (0, 0))],
        out_specs=pl.BlockSpec((4, mb, H), lambda i: (0, i, 0)),
        out_shape=jax.ShapeDtypeStruct((4, E, H), jnp.float32),
    )(pgs, pgd, q, a, geo, wd, b1.reshape(1, H), w2, b2.reshape(1, 2 * H))


def _nodeupd_body(s_ref, ag_ref, v_ref, av_ref, u1_ref, u1b_ref, u2_ref,
                  u2b_ref, g1_ref, g1b_ref, g2_ref, g2b_ref, lng_ref,
                  lnb_ref, so_ref, vo_ref):
    s = s_ref[...]
    ags = ag_ref[...]
    v0 = v_ref[0]; v1 = v_ref[1]; v2 = v_ref[2]
    vnorm = jnp.sqrt(v0 * v0 + v1 * v1 + v2 * v2 + 1e-12)
    ctx = jnp.concatenate([s, ags, vnorm], axis=1)
    hu = _silu(jnp.dot(ctx, u1_ref[...], preferred_element_type=jnp.float32) + u1b_ref[...])
    du = jnp.dot(hu, u2_ref[...], preferred_element_type=jnp.float32) + u2b_ref[...]
    s2 = s + du
    hg = _silu(jnp.dot(ctx, g1_ref[...], preferred_element_type=jnp.float32) + g1b_ref[...])
    gate = _sigmoid(jnp.dot(hg, g2_ref[...], preferred_element_type=jnp.float32) + g2b_ref[...])
    vo_ref[0] = v0 + gate * av_ref[0]
    vo_ref[1] = v1 + gate * av_ref[1]
    vo_ref[2] = v2 + gate * av_ref[2]
    mu = jnp.mean(s2, axis=1, keepdims=True)
    d = s2 - mu
    var = jnp.mean(d * d, axis=1, keepdims=True)
    y = d / jnp.sqrt(var + 1e-5) * lng_ref[...] + lnb_ref[...]
    so_ref[...] = _silu(y)


def _nodeupd_call(s, ags, v, av, u1, u1b, u2, u2b, g1, g1b, g2, g2b,
                  lng, lnb, mb=1000):
    grid = N // mb
    row = lambda i: (i, 0)
    full = lambda shape: pl.BlockSpec(shape, lambda i: (0, 0))
    return pl.pallas_call(
        _nodeupd_body,
        grid=(grid,),
        in_specs=[pl.BlockSpec((mb, H), row),
                  pl.BlockSpec((mb, H), row),
                  pl.BlockSpec((3, mb, H), lambda i: (0, i, 0)),
                  pl.BlockSpec((3, mb, H), lambda i: (0, i, 0)),
                  full((3 * H, H)), full((1, H)),
                  full((H, H)), full((1, H)),
                  full((3 * H, H)), full((1, H)),
                  full((H, H)), full((1, H)),
                  full((1, H)), full((1, H))],
        out_specs=[pl.BlockSpec((mb, H), row),
                   pl.BlockSpec((3, mb, H), lambda i: (0, i, 0))],
        out_shape=[jax.ShapeDtypeStruct((N, H), jnp.float32),
                   jax.ShapeDtypeStruct((3, N, H), jnp.float32)],
    )(s, ags, v, av, u1, u1b.reshape(1, H), u2, u2b.reshape(1, H),
      g1, g1b.reshape(1, H), g2, g2b.reshape(1, H),
      lng.reshape(1, H), lnb.reshape(1, H))


def _segsum_body(b_ref, s_ref, o_ref, c_ref):
    i = pl.program_id(0)

    @pl.when(i == 0)
    def _():
        o_ref[...] = jnp.zeros_like(o_ref)
        c_ref[...] = jnp.zeros_like(c_ref)

    b = b_ref[0, 0, :]
    onehot = (b[:, None] == jax.lax.broadcasted_iota(jnp.int32, (1, G), 1)).astype(jnp.float32)
    contrib = jax.lax.dot_general(onehot, s_ref[...], (((0,), (0,)), ((), ())),
                                  preferred_element_type=jnp.float32)
    o_ref[...] += contrib
    c_ref[...] += jnp.broadcast_to(jnp.sum(onehot, axis=0)[:, None], (G, H))


def _segsum_call(batch3, s, mb=1000):
    grid = N // mb
    return pl.pallas_call(
        _segsum_body,
        grid=(grid,),
        in_specs=[pl.BlockSpec((1, 1, mb), lambda i: (i, 0, 0)),
                  pl.BlockSpec((mb, H), lambda i: (i, 0))],
        out_specs=[pl.BlockSpec((G, H), lambda i: (0, 0)),
                   pl.BlockSpec((G, H), lambda i: (0, 0))],
        out_shape=[jax.ShapeDtypeStruct((G, H), jnp.float32),
                   jax.ShapeDtypeStruct((G, H), jnp.float32)],
    )(batch3, s)


def _bcastadd_body(b_ref, s_ref, t_ref, o_ref):
    b = b_ref[0, 0, :]
    onehot = (b[:, None] == jax.lax.broadcasted_iota(jnp.int32, (1, G), 1)).astype(jnp.float32)
    o_ref[...] = s_ref[...] + jnp.dot(onehot, t_ref[...], preferred_element_type=jnp.float32)


def _bcastadd_call(batch3, s, tbl, mb=1000):
    grid = N // mb
    return pl.pallas_call(
        _bcastadd_body,
        grid=(grid,),
        in_specs=[pl.BlockSpec((1, 1, mb), lambda i: (i, 0, 0)),
                  pl.BlockSpec((mb, H), lambda i: (i, 0)),
                  pl.BlockSpec((G, H), lambda i: (0, 0))],
        out_specs=pl.BlockSpec((mb, H), lambda i: (i, 0)),
        out_shape=jax.ShapeDtypeStruct((N, H), jnp.float32),
    )(batch3, s, tbl)


def _bm_body(a_ref, b_ref, o_ref):
    o_ref[...] = jnp.maximum(a_ref[...] + b_ref[...], 0.0)


def _bm_call(a, b, mb=2000):
    M = a.shape[0]
    grid = M // mb
    return pl.pallas_call(
        _bm_body,
        grid=(grid,),
        in_specs=[pl.BlockSpec((mb, H), lambda i: (i, 0)),
                  pl.BlockSpec((mb, H), lambda i: (i, 0))],
        out_specs=pl.BlockSpec((mb, H), lambda i: (i, 0)),
        out_shape=jax.ShapeDtypeStruct((M, H), jnp.float32),
    )(a, b)


def _add_body(a_ref, b_ref, o_ref):
    o_ref[...] = a_ref[...] + b_ref[...]


def _add_call(a, b, mb=1000):
    M = a.shape[0]
    grid = M // mb
    return pl.pallas_call(
        _add_body,
        grid=(grid,),
        in_specs=[pl.BlockSpec((mb, H), lambda i: (i, 0)),
                  pl.BlockSpec((mb, H), lambda i: (i, 0))],
        out_specs=pl.BlockSpec((mb, H), lambda i: (i, 0)),
        out_shape=jax.ShapeDtypeStruct((M, H), jnp.float32),
    )(a, b)


def _lnsilu_body(x_ref, g_ref, b_ref, o_ref):
    x = x_ref[...]
    mu = jnp.mean(x, axis=1, keepdims=True)
    d = x - mu
    var = jnp.mean(d * d, axis=1, keepdims=True)
    y = d / jnp.sqrt(var + 1e-5) * g_ref[...] + b_ref[...]
    o_ref[...] = _silu(y)


def _lnsilu_call(x, g, b, mb=1000):
    M = x.shape[0]
    grid = M // mb
    return pl.pallas_call(
        _lnsilu_body,
        grid=(grid,),
        in_specs=[pl.BlockSpec((mb, H), lambda i: (i, 0)),
                  pl.BlockSpec((1, H), lambda i: (0, 0)),
                  pl.BlockSpec((1, H), lambda i: (0, 0))],
        out_specs=pl.BlockSpec((mb, H), lambda i: (i, 0)),
        out_shape=jax.ShapeDtypeStruct((M, H), jnp.float32),
    )(x, g.reshape(1, H), b.reshape(1, H))


def kernel(x, pos, edge_index, edge_index_bond, edge_attr_bond, triplet_kj, triplet_ji, batch, params):
    P = params
    src = edge_index[0]; dst = edge_index[1]
    bsrc = edge_index_bond[0]; bdst = edge_index_bond[1]

    order = jnp.argsort(triplet_ji)
    ji_s = triplet_ji[order]
    kj_s = triplet_kj[order]

    batch3 = batch.reshape(10, 1, 1000)

    pp = jnp.concatenate([pos[src], jnp.zeros((E, 1), jnp.float32),
                          pos[dst], jnp.zeros((E, 1), jnp.float32)], axis=1)
    geo = _geo_call(pp)

    gk = _sc_gather(geo, kj_s, gb=128)
    gj = _sc_gather(geo, ji_s, gb=128)
    tfeat = _tripfeat_call(gk, gj)

    s = _mlp_call(x, P['Win'], P['bin'], mb=1000)
    ee = _mlp_call(edge_attr_bond, P['bgWe'], P['bgbe'], mb=2000)
    sg = _sc_gather(s, bsrc, gb=128)
    bm = _bm_call(sg, ee)
    bagg = jnp.zeros((N, H), jnp.float32).at[bdst].add(bm)
    xin = _add_call(s, bagg)
    s = _mlp_call(xin, P['bgW1'], P['bgb1'], P['bgW2'], P['bgb2'],
                  act1='silu', resid=s, mb=1000)
    s = _lnsilu_call(s, P['bglng'], P['bglnb'])

    v = jnp.zeros((3, N, H), jnp.float32)
    vn = jnp.broadcast_to(P['vn0'], (G, H)).astype(jnp.float32)

    for i in range(L):
        m1W = P['m1W%d' % i]
        m1Wa = m1W[0:H]; m1Wb = m1W[H:2 * H]
        m1Wc = m1W[2 * H:2 * H + RB]; m1Wd = m1W[2 * H + RB:]

        vnc = _mlp_call(vn, P['vnaW1'], P['vnab1'], P['vnaW2'], P['vnab2'],
                        act1='silu', mb=G)
        s = _bcastadd_call(batch3, s, vnc)

        pab = _mlp_call(s, jnp.concatenate([m1Wa, m1Wb], axis=1),
                        jnp.zeros((2 * H,), jnp.float32), mb=1000)
        q = _mlp_call(geo, m1Wc, jnp.zeros((H,), jnp.float32),
                      k_lo=0, k_hi=RB, mb=2000)
        t_msg = _mlp_call(tfeat, P['t1W%d' % i], P['t1b%d' % i],
                          P['t2W%d' % i], P['t2b%d' % i], act1='silu', mb=2000)
        A = jnp.zeros((E, H), jnp.float32).at[ji_s].add(t_msg)

        pgs = _sc_gather(pab[:, :H], src, gb=128)
        pgd = _sc_gather(pab[:, H:], dst, gb=128)
        planes = _edge_call(pgs, pgd, q, A, geo, m1Wd, P['m1b%d' % i],
                            P['m2W%d' % i], P['m2b%d' % i])

        ags = jnp.zeros((N, H), jnp.float32).at[dst].add(planes[0])
        av = jnp.zeros((3, N, H), jnp.float32).at[:, dst].add(planes[1:])

        s, v = _nodeupd_call(s, ags, v, av,
                             P['u1W%d' % i], P['u1b%d' % i],
                             P['u2W%d' % i], P['u2b%d' % i],
                             P['g1W%d' % i], P['g1b%d' % i],
                             P['g2W%d' % i], P['g2b%d' % i],
                             P['lng%d' % i], P['lnb%d' % i])

        gsum, _ = _segsum_call(batch3, s)
        vn = _mlp_call(gsum, P['vnbW1'], P['vnbb1'], P['vnbW2'], P['vnbb2'],
                       act1='silu', resid=vn, mb=G)

    gsum, cnt = _segsum_call(batch3, s)
    mean_g = gsum / jnp.maximum(cnt, 1.0)
    gg = jnp.concatenate([gsum, mean_g], axis=1)
    out = _mlp_call(gg, P['h1W'], P['h1b'], P['h2W'], P['h2b'],
                    act1='silu', mb=G)
    return out


# speedup vs baseline: 1.0273x; 1.0224x over previous
import functools
import jax
import jax.numpy as jnp
import numpy as np
from jax import lax
from jax.experimental import pallas as pl
from jax.experimental.pallas import tpu as pltpu
from jax.experimental.pallas import tpu_sc as plsc

N = 10000; E = 320000; EB = 40000; T = 640000
H = 128; IN = 128; RB = 32; AB = 16; L = 4; G = 64

_ASIN_C = [4.2163199048e-2, 2.4181311049e-2, 4.5470025998e-2,
           7.4953002686e-2, 1.6666752422e-1]


def _acos(x):
    t = jnp.abs(x)
    w = jnp.sqrt(jnp.maximum((1.0 - t) * 0.5, 0.0))
    u = jnp.where(t <= 0.5, t, w)
    z = u * u
    p = jnp.full_like(z, _ASIN_C[0])
    for cf in _ASIN_C[1:]:
        p = p * z + cf
    asin_u = u + u * z * p
    r = jnp.where(t <= 0.5, np.float32(np.pi / 2) - asin_u, 2.0 * asin_u)
    return jnp.where(x >= 0.0, r, np.float32(np.pi) - r)


def _sigmoid(x):
    return jax.nn.sigmoid(x)


def _silu(x):
    return x * _sigmoid(x)


def _sc_gather(table, idx, gb=80):
    B, = idx.shape
    R, D = table.shape
    NW = 32
    chunk = NW * gb
    Bp = ((B + chunk - 1) // chunk) * chunk
    if Bp != B:
        idx = jnp.concatenate([idx, jnp.zeros((Bp - B,), idx.dtype)])
    bt = Bp // NW
    nit = bt // gb
    mesh = plsc.VectorSubcoreMesh(core_axis_name="c", subcore_axis_name="s")

    @functools.partial(
        pl.kernel, mesh=mesh,
        out_type=jax.ShapeDtypeStruct((Bp, D), jnp.float32),
        scratch_types=[pltpu.VMEM((gb,), jnp.int32),
                       pltpu.VMEM((gb, D), jnp.float32),
                       pltpu.SemaphoreType.DMA],
    )
    def k(table_hbm, idx_hbm, out_hbm, idx_v, rows_v, sem):
        wid = lax.axis_index("s") * 2 + lax.axis_index("c")
        base = wid * bt

        def body(it, carry):
            off = base + it * gb
            pltpu.sync_copy(idx_hbm.at[pl.ds(off, gb)], idx_v)
            pltpu.async_copy(table_hbm.at[idx_v], rows_v, sem).wait()
            pltpu.sync_copy(rows_v, out_hbm.at[pl.ds(off, gb)])
            return carry

        lax.fori_loop(0, nit, body, 0)

    out = k(table, idx)
    return out[:B] if Bp != B else out


def _mlp_body(has2, act1, has_resid, k_lo, k_hi, *refs):
    if has2 and has_resid:
        x_ref, w1_ref, b1_ref, w2_ref, b2_ref, r_ref, o_ref = refs
    elif has2:
        x_ref, w1_ref, b1_ref, w2_ref, b2_ref, o_ref = refs
        r_ref = None
    elif has_resid:
        x_ref, w1_ref, b1_ref, r_ref, o_ref = refs
    else:
        x_ref, w1_ref, b1_ref, o_ref = refs
        r_ref = None
    x = x_ref[...]
    if k_lo != 0 or k_hi != x.shape[1]:
        x = x[:, k_lo:k_hi]
    h = jnp.dot(x, w1_ref[...], preferred_element_type=jnp.float32) + b1_ref[...]
    if act1 == 'silu':
        h = _silu(h)
    elif act1 == 'relu':
        h = jnp.maximum(h, 0.0)
    if has2:
        h = jnp.dot(h, w2_ref[...], preferred_element_type=jnp.float32) + b2_ref[...]
    if r_ref is not None:
        h = h + r_ref[...]
    o_ref[...] = h


def _mlp_call(X, W1, b1, W2=None, b2=None, act1=None, resid=None,
              k_lo=0, k_hi=None, mb=2000):
    M, K = X.shape
    if k_hi is None:
        k_hi = K
    Hout = W2.shape[1] if W2 is not None else W1.shape[1]
    H1 = W1.shape[1]
    grid = M // mb
    assert grid * mb == M, (M, mb)
    ins = [X, W1.reshape(W1.shape), b1.reshape(1, H1)]
    specs = [pl.BlockSpec((mb, K), lambda i: (i, 0)),
             pl.BlockSpec(W1.shape, lambda i: (0, 0)),
             pl.BlockSpec((1, H1), lambda i: (0, 0))]
    if W2 is not None:
        ins += [W2, b2.reshape(1, Hout)]
        specs += [pl.BlockSpec(W2.shape, lambda i: (0, 0)),
                  pl.BlockSpec((1, Hout), lambda i: (0, 0))]
    if resid is not None:
        ins.append(resid)
        specs.append(pl.BlockSpec((mb, Hout), lambda i: (i, 0)))
    body = functools.partial(_mlp_body, W2 is not None, act1,
                             resid is not None, k_lo, k_hi)
    return pl.pallas_call(
        body,
        grid=(grid,),
        in_specs=specs,
        out_specs=pl.BlockSpec((mb, Hout), lambda i: (i, 0)),
        out_shape=jax.ShapeDtypeStruct((M, Hout), jnp.float32),
    )(*ins)


def _geo_body(pp_ref, o_ref):
    pp = pp_ref[...]
    rel = pp[:, 4:7] - pp[:, 0:3]
    d2 = jnp.sum(rel * rel, axis=1, keepdims=True) + 1e-12
    dist = jnp.sqrt(d2)
    edir = rel / dist
    delta = 5.0 / (RB - 1)
    gamma = 1.0 / (2.0 * (delta ** 2 + 1e-12))
    c = jax.lax.broadcasted_iota(jnp.int32, (1, RB), 1).astype(jnp.float32) * np.float32(delta)
    diff = dist - c
    erbf = jnp.exp(-gamma * diff * diff)
    pad = jnp.zeros((pp.shape[0], 93), jnp.float32)
    o_ref[...] = jnp.concatenate([erbf, edir, pad], axis=1)


def _geo_call(pp, mb=2000):
    grid = E // mb
    return pl.pallas_call(
        _geo_body,
        grid=(grid,),
        in_specs=[pl.BlockSpec((mb, 8), lambda i: (i, 0))],
        out_specs=pl.BlockSpec((mb, 128), lambda i: (i, 0)),
        out_shape=jax.ShapeDtypeStruct((E, 128), jnp.float32),
    )(pp)


def _tripfeat_body(gk_ref, gj_ref, o_ref):
    gk = gk_ref[...]
    gj = gj_ref[...]
    cos = jnp.sum(gk[:, 32:35] * gj[:, 32:35], axis=1, keepdims=True)
    cos = jnp.clip(cos, -1.0 + 1e-7, 1.0 - 1e-7)
    theta = _acos(cos)
    delta = float(np.pi) / (AB - 1)
    gamma = 1.0 / (2.0 * (delta ** 2 + 1e-12))
    c = jax.lax.broadcasted_iota(jnp.int32, (1, AB), 1).astype(jnp.float32) * np.float32(delta)
    diff = theta - c
    af = jnp.exp(-gamma * diff * diff)
    o_ref[...] = jnp.concatenate([gk[:, 0:32], gj[:, 0:32], af], axis=1)


def _tripfeat_call(gk, gj, mb=2000):
    grid = T // mb
    return pl.pallas_call(
        _tripfeat_body,
        grid=(grid,),
        in_specs=[pl.BlockSpec((mb, 128), lambda i: (i, 0)),
                  pl.BlockSpec((mb, 128), lambda i: (i, 0))],
        out_specs=pl.BlockSpec((mb, 80), lambda i: (i, 0)),
        out_shape=jax.ShapeDtypeStruct((T, 80), jnp.float32),
    )(gk, gj)


def _edge_body(pgs_ref, pgd_ref, q_ref, a_ref, geo_ref, wd_ref, b1_ref,
               w2_ref, b2_ref, o_ref):
    adot = jnp.dot(a_ref[...], wd_ref[...], preferred_element_type=jnp.float32)
    h1 = pgs_ref[...] + pgd_ref[...] + q_ref[...] + adot + b1_ref[...]
    m = jnp.dot(_silu(h1), w2_ref[...], preferred_element_type=jnp.float32) + b2_ref[...]
    geo = geo_ref[...]
    m_s = m[:, :H]
    m_vc = m[:, H:]
    o_ref[0] = m_s
    o_ref[1] = m_vc * geo[:, 32:33]
    o_ref[2] = m_vc * geo[:, 33:34]
    o_ref[3] = m_vc * geo[:, 34:35]


def _edge_call(pgs, pgd, q, a, geo, wd, b1, w2, b2, mb=2000):
    grid = E // mb
    return pl.pallas_call(
        _edge_body,
        grid=(grid,),
        in_specs=[pl.BlockSpec((mb, H), lambda i: (i, 0)),
                  pl.BlockSpec((mb, H), lambda i: (i, 0)),
                  pl.BlockSpec((mb, H), lambda i: (i, 0)),
                  pl.BlockSpec((mb, H), lambda i: (i, 0)),
                  pl.BlockSpec((mb, 128), lambda i: (i, 0)),
                  pl.BlockSpec((H, H), lambda i: (0, 0)),
                  pl.BlockSpec((1, H), lambda i: (0, 0)),
                  pl.BlockSpec((H, 2 * H), lambda i: (0, 0)),
                  pl.BlockSpec((1, 2 * H), lambda i: (0, 0))],
        out_specs=pl.BlockSpec((4, mb, H), lambda i: (0, i, 0)),
        out_shape=jax.ShapeDtypeStruct((4, E, H), jnp.float32),
    )(pgs, pgd, q, a, geo, wd, b1.reshape(1, H), w2, b2.reshape(1, 2 * H))


def _nodeupd_body(s_ref, ag_ref, v_ref, av_ref, u1_ref, u1b_ref, u2_ref,
                  u2b_ref, g1_ref, g1b_ref, g2_ref, g2b_ref, lng_ref,
                  lnb_ref, so_ref, vo_ref):
    s = s_ref[...]
    ags = ag_ref[...]
    v0 = v_ref[0]; v1 = v_ref[1]; v2 = v_ref[2]
    vnorm = jnp.sqrt(v0 * v0 + v1 * v1 + v2 * v2 + 1e-12)
    ctx = jnp.concatenate([s, ags, vnorm], axis=1)
    hu = _silu(jnp.dot(ctx, u1_ref[...], preferred_element_type=jnp.float32) + u1b_ref[...])
    du = jnp.dot(hu, u2_ref[...], preferred_element_type=jnp.float32) + u2b_ref[...]
    s2 = s + du
    hg = _silu(jnp.dot(ctx, g1_ref[...], preferred_element_type=jnp.float32) + g1b_ref[...])
    gate = _sigmoid(jnp.dot(hg, g2_ref[...], preferred_element_type=jnp.float32) + g2b_ref[...])
    vo_ref[0] = v0 + gate * av_ref[0]
    vo_ref[1] = v1 + gate * av_ref[1]
    vo_ref[2] = v2 + gate * av_ref[2]
    mu = jnp.mean(s2, axis=1, keepdims=True)
    d = s2 - mu
    var = jnp.mean(d * d, axis=1, keepdims=True)
    y = d / jnp.sqrt(var + 1e-5) * lng_ref[...] + lnb_ref[...]
    so_ref[...] = _silu(y)


def _nodeupd_call(s, ags, v, av, u1, u1b, u2, u2b, g1, g1b, g2, g2b,
                  lng, lnb, mb=1000):
    grid = N // mb
    row = lambda i: (i, 0)
    full = lambda shape: pl.BlockSpec(shape, lambda i: (0, 0))
    return pl.pallas_call(
        _nodeupd_body,
        grid=(grid,),
        in_specs=[pl.BlockSpec((mb, H), row),
                  pl.BlockSpec((mb, H), row),
                  pl.BlockSpec((3, mb, H), lambda i: (0, i, 0)),
                  pl.BlockSpec((3, mb, H), lambda i: (0, i, 0)),
                  full((3 * H, H)), full((1, H)),
                  full((H, H)), full((1, H)),
                  full((3 * H, H)), full((1, H)),
                  full((H, H)), full((1, H)),
                  full((1, H)), full((1, H))],
        out_specs=[pl.BlockSpec((mb, H), row),
                   pl.BlockSpec((3, mb, H), lambda i: (0, i, 0))],
        out_shape=[jax.ShapeDtypeStruct((N, H), jnp.float32),
                   jax.ShapeDtypeStruct((3, N, H), jnp.float32)],
    )(s, ags, v, av, u1, u1b.reshape(1, H), u2, u2b.reshape(1, H),
      g1, g1b.reshape(1, H), g2, g2b.reshape(1, H),
      lng.reshape(1, H), lnb.reshape(1, H))


def _segsum_body(b_ref, s_ref, o_ref, c_ref):
    i = pl.program_id(0)

    @pl.when(i == 0)
    def _():
        o_ref[...] = jnp.zeros_like(o_ref)
        c_ref[...] = jnp.zeros_like(c_ref)

    b = b_ref[0, 0, :]
    onehot = (b[:, None] == jax.lax.broadcasted_iota(jnp.int32, (1, G), 1)).astype(jnp.float32)
    contrib = jax.lax.dot_general(onehot, s_ref[...], (((0,), (0,)), ((), ())),
                                  preferred_element_type=jnp.float32)
    o_ref[...] += contrib
    c_ref[...] += jnp.broadcast_to(jnp.sum(onehot, axis=0)[:, None], (G, H))


def _segsum_call(batch3, s, mb=1000):
    grid = N // mb
    return pl.pallas_call(
        _segsum_body,
        grid=(grid,),
        in_specs=[pl.BlockSpec((1, 1, mb), lambda i: (i, 0, 0)),
                  pl.BlockSpec((mb, H), lambda i: (i, 0))],
        out_specs=[pl.BlockSpec((G, H), lambda i: (0, 0)),
                   pl.BlockSpec((G, H), lambda i: (0, 0))],
        out_shape=[jax.ShapeDtypeStruct((G, H), jnp.float32),
                   jax.ShapeDtypeStruct((G, H), jnp.float32)],
    )(batch3, s)


def _bcastadd_body(b_ref, s_ref, t_ref, o_ref):
    b = b_ref[0, 0, :]
    onehot = (b[:, None] == jax.lax.broadcasted_iota(jnp.int32, (1, G), 1)).astype(jnp.float32)
    o_ref[...] = s_ref[...] + jnp.dot(onehot, t_ref[...], preferred_element_type=jnp.float32)


def _bcastadd_call(batch3, s, tbl, mb=1000):
    grid = N // mb
    return pl.pallas_call(
        _bcastadd_body,
        grid=(grid,),
        in_specs=[pl.BlockSpec((1, 1, mb), lambda i: (i, 0, 0)),
                  pl.BlockSpec((mb, H), lambda i: (i, 0)),
                  pl.BlockSpec((G, H), lambda i: (0, 0))],
        out_specs=pl.BlockSpec((mb, H), lambda i: (i, 0)),
        out_shape=jax.ShapeDtypeStruct((N, H), jnp.float32),
    )(batch3, s, tbl)


def _bm_body(a_ref, b_ref, o_ref):
    o_ref[...] = jnp.maximum(a_ref[...] + b_ref[...], 0.0)


def _bm_call(a, b, mb=2000):
    M = a.shape[0]
    grid = M // mb
    return pl.pallas_call(
        _bm_body,
        grid=(grid,),
        in_specs=[pl.BlockSpec((mb, H), lambda i: (i, 0)),
                  pl.BlockSpec((mb, H), lambda i: (i, 0))],
        out_specs=pl.BlockSpec((mb, H), lambda i: (i, 0)),
        out_shape=jax.ShapeDtypeStruct((M, H), jnp.float32),
    )(a, b)


def _add_body(a_ref, b_ref, o_ref):
    o_ref[...] = a_ref[...] + b_ref[...]


def _add_call(a, b, mb=1000):
    M = a.shape[0]
    grid = M // mb
    return pl.pallas_call(
        _add_body,
        grid=(grid,),
        in_specs=[pl.BlockSpec((mb, H), lambda i: (i, 0)),
                  pl.BlockSpec((mb, H), lambda i: (i, 0))],
        out_specs=pl.BlockSpec((mb, H), lambda i: (i, 0)),
        out_shape=jax.ShapeDtypeStruct((M, H), jnp.float32),
    )(a, b)


def _lnsilu_body(x_ref, g_ref, b_ref, o_ref):
    x = x_ref[...]
    mu = jnp.mean(x, axis=1, keepdims=True)
    d = x - mu
    var = jnp.mean(d * d, axis=1, keepdims=True)
    y = d / jnp.sqrt(var + 1e-5) * g_ref[...] + b_ref[...]
    o_ref[...] = _silu(y)


def _lnsilu_call(x, g, b, mb=1000):
    M = x.shape[0]
    grid = M // mb
    return pl.pallas_call(
        _lnsilu_body,
        grid=(grid,),
        in_specs=[pl.BlockSpec((mb, H), lambda i: (i, 0)),
                  pl.BlockSpec((1, H), lambda i: (0, 0)),
                  pl.BlockSpec((1, H), lambda i: (0, 0))],
        out_specs=pl.BlockSpec((mb, H), lambda i: (i, 0)),
        out_shape=jax.ShapeDtypeStruct((M, H), jnp.float32),
    )(x, g.reshape(1, H), b.reshape(1, H))


def kernel(x, pos, edge_index, edge_index_bond, edge_attr_bond, triplet_kj, triplet_ji, batch, params):
    P = params
    src = edge_index[0]; dst = edge_index[1]
    bsrc = edge_index_bond[0]; bdst = edge_index_bond[1]

    order = jnp.argsort(triplet_ji)
    ji_s = triplet_ji[order]
    kj_s = triplet_kj[order]

    batch3 = batch.reshape(10, 1, 1000)

    pp = jnp.concatenate([pos[src], jnp.zeros((E, 1), jnp.float32),
                          pos[dst], jnp.zeros((E, 1), jnp.float32)], axis=1)
    geo = _geo_call(pp)

    gk = _sc_gather(geo, kj_s, gb=128)
    gj = _sc_gather(geo, ji_s, gb=128)
    tfeat = _tripfeat_call(gk, gj)

    s = _mlp_call(x, P['Win'], P['bin'], mb=1000)
    ee = _mlp_call(edge_attr_bond, P['bgWe'], P['bgbe'], mb=2000)
    sg = _sc_gather(s, bsrc, gb=128)
    bm = _bm_call(sg, ee)
    bagg = jnp.zeros((N, H), jnp.float32).at[bdst].add(bm)
    xin = _add_call(s, bagg)
    s = _mlp_call(xin, P['bgW1'], P['bgb1'], P['bgW2'], P['bgb2'],
                  act1='silu', resid=s, mb=1000)
    s = _lnsilu_call(s, P['bglng'], P['bglnb'])

    v = jnp.zeros((3, N, H), jnp.float32)
    vn = jnp.broadcast_to(P['vn0'], (G, H)).astype(jnp.float32)

    for i in range(L):
        m1W = P['m1W%d' % i]
        m1Wa = m1W[0:H]; m1Wb = m1W[H:2 * H]
        m1Wc = m1W[2 * H:2 * H + RB]; m1Wd = m1W[2 * H + RB:]

        vnc = _mlp_call(vn, P['vnaW1'], P['vnab1'], P['vnaW2'], P['vnab2'],
                        act1='silu', mb=G)
        s = _bcastadd_call(batch3, s, vnc)

        pab = _mlp_call(s, jnp.concatenate([m1Wa, m1Wb], axis=1),
                        jnp.zeros((2 * H,), jnp.float32), mb=1000)
        q = _mlp_call(geo, m1Wc, jnp.zeros((H,), jnp.float32),
                      k_lo=0, k_hi=RB, mb=2000)
        t_msg = _mlp_call(tfeat, P['t1W%d' % i], P['t1b%d' % i],
                          P['t2W%d' % i], P['t2b%d' % i], act1='silu', mb=2000)
        A = jnp.zeros((E, H), jnp.float32).at[ji_s].add(t_msg)

        pgs = pab[:, :H][src]
        pgd = pab[:, H:][dst]
        planes = _edge_call(pgs, pgd, q, A, geo, m1Wd, P['m1b%d' % i],
                            P['m2W%d' % i], P['m2b%d' % i])

        ags = jnp.zeros((N, H), jnp.float32).at[dst].add(planes[0])
        av = jnp.zeros((3, N, H), jnp.float32).at[:, dst].add(planes[1:])

        s, v = _nodeupd_call(s, ags, v, av,
                             P['u1W%d' % i], P['u1b%d' % i],
                             P['u2W%d' % i], P['u2b%d' % i],
                             P['g1W%d' % i], P['g1b%d' % i],
                             P['g2W%d' % i], P['g2b%d' % i],
                             P['lng%d' % i], P['lnb%d' % i])

        gsum, _ = _segsum_call(batch3, s)
        vn = _mlp_call(gsum, P['vnbW1'], P['vnbb1'], P['vnbW2'], P['vnbb2'],
                       act1='silu', resid=vn, mb=G)

    gsum, cnt = _segsum_call(batch3, s)
    mean_g = gsum / jnp.maximum(cnt, 1.0)
    gg = jnp.concatenate([gsum, mean_g], axis=1)
    out = _mlp_call(gg, P['h1W'], P['h1b'], P['h2W'], P['h2b'],
                    act1='silu', mb=G)
    return out
